# baseline (speedup 1.0000x reference)
import jax
import jax.numpy as jnp
from jax.experimental import pallas as pl
from jax.experimental.pallas import tpu as pltpu

_B, _N, _E = 32, 128, 16256
_EC = 4064
_NC = _E // _EC
_EPS = 1e-3
_SQRT2 = 1.4142135623730951
_BF = jnp.bfloat16


def _gelu(v):
    return 0.5 * v * (1.0 + jax.lax.erf(v / _SQRT2))


def _bn_scale(p):
    gamma, beta, mm, mv, _, _ = p
    s = gamma / jnp.sqrt(mv + _EPS)
    return s, beta - mm * s


def _bdot(a, w_ref):
    return jnp.dot(a.astype(_BF), w_ref, preferred_element_type=jnp.float32)


def _body(hx_ref, nidx_ref, bidx_ref, seg_ref, tb_ref, bnp_ref, w20_ref,
          wp_ref, wcat_ref, cr_ref, ox_ref, oh_ref, thi_s, tlo_s, acc_s):
    c = pl.program_id(1)
    nc = pl.num_programs(1)

    @pl.when(c == 0)
    def _init():
        hxb = hx_ref[0]
        hxn = hxb * bnp_ref[0:1, 0:20] + bnp_ref[1:2, 0:20]
        hxm = hxb * bnp_ref[2:3, 0:20] + bnp_ref[3:4, 0:20]
        bias1 = tb_ref[0, 0, 0:16][None, :]
        pn = _bdot(hxn, w20_ref[:, 0:16]) + bias1
        pb = _bdot(hxm, w20_ref[:, 16:32])
        zeros4 = jnp.zeros((_N, 4), jnp.float32)
        xb = hxb[:, 16:20]
        tfull = jnp.concatenate(
            [jnp.concatenate([pn, xb, zeros4], axis=1),
             jnp.concatenate([pb, zeros4, xb], axis=1)], axis=0)
        hi = tfull.astype(_BF)
        thi_s[...] = hi
        tlo_s[...] = (tfull - hi.astype(jnp.float32)).astype(_BF)
        acc_s[...] = jnp.zeros((_N, 24), jnp.float32)

    ids_n = nidx_ref[0, 0, 0, :]
    ids_b = bidx_ref[0, 0, 0, :]
    seg = seg_ref[0, :]

    lane = jax.lax.broadcasted_iota(jnp.int32, (_EC, 2 * _N), 1)
    oh = ((ids_n[:, None] == lane) | (ids_b[:, None] == lane)
          ).astype(_BF)

    g = (jnp.dot(oh, thi_s[...], preferred_element_type=jnp.float32)
         + jnp.dot(oh, tlo_s[...], preferred_element_type=jnp.float32))

    z1 = _gelu(g[:, 0:16])
    msg = _gelu(_bdot(z1 * bnp_ref[4:5, 0:16] + bnp_ref[5:6, 0:16],
                      wp_ref[:, 0:16]) + cr_ref[0, 0:16][None, :])
    cfz = _gelu(_bdot(msg * bnp_ref[6:7, 0:16] + bnp_ref[7:8, 0:16],
                      wp_ref[:, 16:32]) + tb_ref[0, 0, 16:32][None, :])
    mc = jnp.concatenate(
        [msg, cfz * bnp_ref[8:9, 0:16] + bnp_ref[9:10, 0:16]], axis=1)
    abc = _gelu(_bdot(mc, wcat_ref[...])
                + tb_ref[0, 0, 48:56][None, :])
    cu = abc[:, 2:3] * (abc[:, 0:1] * g[:, 16:20]
                        + abc[:, 1:2] * g[:, 20:24])

    sub = jax.lax.broadcasted_iota(jnp.int32, (_N, _EC), 0)
    oh_s = (sub == seg).astype(_BF)
    scat = jnp.concatenate(
        [msg, cu, jnp.ones((_EC, 1), jnp.float32),
         jnp.zeros((_EC, 3), jnp.float32)], axis=1)
    shi = scat.astype(_BF)
    slo = (scat - shi.astype(jnp.float32)).astype(_BF)
    acc_s[...] += (jnp.dot(oh_s, shi, preferred_element_type=jnp.float32)
                   + jnp.dot(oh_s, slo, preferred_element_type=jnp.float32))

    @pl.when(c == nc - 1)
    def _fin():
        aggm = acc_s[:, 0:16]
        aggc = acc_s[:, 16:20]
        cnt = acc_s[:, 20:21]
        hxb = hx_ref[0]
        ox_ref[0] = hxb[:, 16:20] + jnp.where(
            cnt > 0.0, aggc / jnp.maximum(cnt, 1.0), 0.0)
        zi = _gelu(
            _bdot(hxb[:, 0:16] * bnp_ref[10:11, 0:16] + bnp_ref[11:12, 0:16],
                  wp_ref[:, 48:64])
            + _bdot(aggm * bnp_ref[12:13, 0:16] + bnp_ref[13:14, 0:16],
                    wp_ref[:, 64:80])
            + tb_ref[0, 0, 32:48][None, :])
        oh_ref[0] = _gelu(
            _bdot(zi * bnp_ref[14:15, 0:16] + bnp_ref[15:16, 0:16],
                  wp_ref[:, 32:48]) + cr_ref[0, 16:32][None, :])


def kernel(x, h, edges, edge_weights, time_embed, message_params,
           coord_params, inv_params, Wa, ba, Wb, bb):
    del edge_weights
    f32 = jnp.float32
    s1, t1 = _bn_scale(message_params[0])
    s2, t2 = _bn_scale(message_params[1])
    sc1, tc1 = _bn_scale(coord_params[0])
    sc2, tc2 = _bn_scale(coord_params[1])
    si1, ti1 = _bn_scale(inv_params[0])
    si2, ti2 = _bn_scale(inv_params[1])
    w1, b1 = message_params[0][4], message_params[0][5]
    w2, b2 = message_params[1][4], message_params[1][5]
    wc1, bc1 = coord_params[0][4], coord_params[0][5]
    wc2, bc2 = coord_params[1][4], coord_params[1][5]
    wi1, bi1 = inv_params[0][4], inv_params[0][5]
    wi2, bi2 = inv_params[1][4], inv_params[1][5]

    def bdot(a, w):
        return jnp.dot(a.astype(_BF), w.astype(_BF),
                       preferred_element_type=f32)

    te1 = time_embed * s1[40:48] + t1[40:48]
    bias1_t = bdot(te1, w1[40:48]) + b1
    tec1 = time_embed * sc1[16:24] + tc1[16:24]
    biasc1_t = bdot(tec1, wc1[16:24]) + bc1
    tei1 = time_embed * si1[32:40] + ti1[32:40]
    biasi_t = bdot(tei1, wi1[32:40]) + bi1
    ca = bdot(time_embed, Wa[16:24]) + ba
    cb = bdot(time_embed, Wb[16:24]) + bb
    bc2_b = jnp.broadcast_to(bc2[None, :], (_B, 1))
    tb = jnp.concatenate(
        [bias1_t, biasc1_t, biasi_t, ca, cb, bc2_b,
         jnp.zeros((_B, 13), f32)], axis=1).reshape(_B, 1, 64)

    def row20(v):
        return jnp.concatenate([v, jnp.zeros((32 - v.shape[0],), f32)])
    bnp = jnp.stack([
        row20(jnp.concatenate([s1[0:16], s1[32:36]])),
        row20(jnp.concatenate([t1[0:16], t1[32:36]])),
        row20(jnp.concatenate([s1[16:32], s1[36:40]])),
        row20(jnp.concatenate([t1[16:32], t1[36:40]])),
        row20(s2), row20(t2),
        row20(sc1[0:16]), row20(tc1[0:16]),
        row20(sc2), row20(tc2),
        row20(si1[0:16]), row20(ti1[0:16]),
        row20(si1[16:32]), row20(ti1[16:32]),
        row20(si2), row20(ti2)])

    w20 = jnp.concatenate(
        [jnp.concatenate([w1[0:16], w1[32:36]], axis=0),
         jnp.concatenate([w1[16:32], w1[36:40]], axis=0)],
        axis=1).astype(_BF)
    wp = jnp.concatenate(
        [w2, wc1[0:16], wi2, wi1[0:16], wi1[16:32],
         jnp.zeros((16, 16), f32)], axis=1).astype(_BF)
    z16 = jnp.zeros((16, 1), f32)
    wcat = jnp.concatenate(
        [jnp.concatenate([Wa[0:16], Wb[0:16], z16], axis=1),
         jnp.concatenate([z16, z16, wc2], axis=1)], axis=0)
    wcat = jnp.concatenate([wcat, jnp.zeros((32, 5), f32)],
                           axis=1).astype(_BF)
    cr = jnp.concatenate([b2, bi2, jnp.zeros((32,), f32)])[None, :]

    hx = jnp.concatenate([h, x], axis=2)
    nidx = edges[:, :, 0].reshape(_B, _NC, 1, _EC)
    bidx = (edges[:, :, 1] + _N).reshape(_B, _NC, 1, _EC)
    seg = edges[0, :, 0].reshape(_NC, 1, _EC)

    grid = (_B, _NC)
    ox, oh = pl.pallas_call(
        _body,
        grid=grid,
        in_specs=[
            pl.BlockSpec((1, _N, 20), lambda b, c: (b, 0, 0)),
            pl.BlockSpec((1, 1, 1, _EC), lambda b, c: (b, c, 0, 0)),
            pl.BlockSpec((1, 1, 1, _EC), lambda b, c: (b, c, 0, 0)),
            pl.BlockSpec((1, 1, _EC), lambda b, c: (c, 0, 0)),
            pl.BlockSpec((1, 1, 64), lambda b, c: (b, 0, 0)),
            pl.BlockSpec((16, 32), lambda b, c: (0, 0)),
            pl.BlockSpec((20, 32), lambda b, c: (0, 0)),
            pl.BlockSpec((16, 96), lambda b, c: (0, 0)),
            pl.BlockSpec((32, 8), lambda b, c: (0, 0)),
            pl.BlockSpec((1, 64), lambda b, c: (0, 0)),
        ],
        out_specs=[
            pl.BlockSpec((1, _N, 4), lambda b, c: (b, 0, 0)),
            pl.BlockSpec((1, _N, 16), lambda b, c: (b, 0, 0)),
        ],
        out_shape=[
            jax.ShapeDtypeStruct((_B, _N, 4), f32),
            jax.ShapeDtypeStruct((_B, _N, 16), f32),
        ],
        scratch_shapes=[
            pltpu.VMEM((2 * _N, 24), _BF),
            pltpu.VMEM((2 * _N, 24), _BF),
            pltpu.VMEM((_N, 24), f32),
        ],
        compiler_params=pltpu.CompilerParams(
            dimension_semantics=("arbitrary", "arbitrary")),
    )(hx, nidx, bidx, seg, tb, bnp, w20, wp, wcat, cr)
    return (ox, oh)
